# baseline (speedup 1.0000x reference)
import functools

import jax
import jax.numpy as jnp
from jax import lax
from jax.experimental import pallas as pl
from jax.experimental.pallas import tpu as pltpu
from jax.experimental.pallas import tpu_sc as plsc

_INFO = plsc.get_sparse_core_info()
_NC, _NS = _INFO.num_cores, _INFO.num_subcores
_NW = _NC * _NS

_CHUNK = 100
_NBUF = 8


def _make_kernel(n_chunks, embed):
    per_w = n_chunks // _NW

    mesh = plsc.VectorSubcoreMesh(core_axis_name="c", subcore_axis_name="s")

    @functools.partial(
        pl.kernel,
        out_type=jax.ShapeDtypeStruct((n_chunks, _CHUNK, embed), jnp.float32),
        mesh=mesh,
        scratch_types=[
            pltpu.VMEM((per_w, _CHUNK), jnp.int32),
            pltpu.VMEM((per_w, _CHUNK), jnp.int32),
            pltpu.VMEM((_NBUF, _CHUNK, embed), jnp.float32),
        ]
        + [pltpu.SemaphoreType.DMA] * _NBUF,
        compiler_params=pltpu.CompilerParams(use_tc_tiling_on_sc=False),
    )
    def k(x_hbm, tok_hbm, pos_hbm, pidx_hbm, out_hbm, idx_v, pidx_v, rows_v,
          *sems):
        wid = lax.axis_index("s") * _NC + lax.axis_index("c")
        base = wid * per_w
        pltpu.sync_copy(x_hbm.at[wid], idx_v)
        pltpu.sync_copy(pidx_hbm.at[wid], pidx_v)

        @pl.loop(0, per_w, step=_NBUF)
        def group(g):
            toks = []
            for b in range(_NBUF):
                toks.append(pltpu.async_copy(
                    tok_hbm.at[idx_v.at[g + b]], rows_v.at[b], sems[b]))
            poss = []
            for b in range(_NBUF):
                toks[b].wait()
                poss.append(pltpu.async_copy(
                    pos_hbm.at[pidx_v.at[g + b]], rows_v.at[b], sems[b],
                    add=True))
            outs = []
            for b in range(_NBUF):
                poss[b].wait()
                outs.append(pltpu.async_copy(
                    rows_v.at[b], out_hbm.at[base + g + b], sems[b]))
            for b in range(_NBUF):
                outs[b].wait()

    return k


def kernel(x, token_table, pos_table):
    batch, maxlen = x.shape
    embed = token_table.shape[-1]
    n_flat = batch * maxlen
    n_chunks = n_flat // _CHUNK
    xt = jnp.transpose(x).astype(jnp.int32).reshape(_NW, n_chunks // _NW, _CHUNK)
    pidx = (jnp.arange(n_flat, dtype=jnp.int32) // batch).reshape(
        _NW, n_chunks // _NW, _CHUNK)
    k = _make_kernel(n_chunks, embed)
    out = k(xt, token_table, pos_table, pidx)
    return jnp.transpose(out.reshape(maxlen, batch, embed), (1, 0, 2))


# speedup vs baseline: 1.1986x; 1.1986x over previous
import functools

import jax
import jax.numpy as jnp
from jax import lax
from jax.experimental import pallas as pl
from jax.experimental.pallas import tpu as pltpu
from jax.experimental.pallas import tpu_sc as plsc

_INFO = plsc.get_sparse_core_info()
_NC, _NS = _INFO.num_cores, _INFO.num_subcores
_NW = _NC * _NS

_CHUNK = 100
_NBUF = 8


def _make_kernel(n_chunks, embed):
    per_w = n_chunks // _NW

    mesh = plsc.VectorSubcoreMesh(core_axis_name="c", subcore_axis_name="s")

    @functools.partial(
        pl.kernel,
        out_type=jax.ShapeDtypeStruct((n_chunks, _CHUNK, embed), jnp.float32),
        mesh=mesh,
        scratch_types=[
            pltpu.VMEM((per_w, _CHUNK), jnp.int32),
            pltpu.VMEM((2, _CHUNK), jnp.int32),
            pltpu.VMEM((_NBUF, _CHUNK, embed), jnp.float32),
        ]
        + [pltpu.SemaphoreType.DMA] * _NBUF,
        compiler_params=pltpu.CompilerParams(use_tc_tiling_on_sc=False),
    )
    def k(x_hbm, tok_hbm, pos_hbm, pidx_hbm, out_hbm, idx_v, pidx_v, rows_v,
          *sems):
        wid = lax.axis_index("s") * _NC + lax.axis_index("c")
        base = wid * per_w
        pltpu.sync_copy(x_hbm.at[wid], idx_v)
        pltpu.sync_copy(pidx_hbm, pidx_v)

        @pl.loop(0, per_w, step=_NBUF)
        def group(g):
            toks = []
            for b in range(_NBUF):
                toks.append(pltpu.async_copy(
                    tok_hbm.at[idx_v.at[g + b]], rows_v.at[b], sems[b]))
            poss = []
            for b in range(_NBUF):
                toks[b].wait()
                poss.append(pltpu.async_copy(
                    pos_hbm.at[pidx_v.at[b % 2]], rows_v.at[b], sems[b],
                    add=True))
            outs = []
            for b in range(_NBUF):
                poss[b].wait()
                outs.append(pltpu.async_copy(
                    rows_v.at[b], out_hbm.at[base + g + b], sems[b]))
            for b in range(_NBUF):
                outs[b].wait()

    return k


def kernel(x, token_table, pos_table):
    batch, maxlen = x.shape
    embed = token_table.shape[-1]
    n_flat = batch * maxlen
    n_chunks = n_flat // _CHUNK
    xt = x.astype(jnp.int32).reshape(_NW, n_chunks // _NW, _CHUNK)
    pidx = jnp.arange(maxlen, dtype=jnp.int32).reshape(2, _CHUNK)
    k = _make_kernel(n_chunks, embed)
    out = k(xt, token_table, pos_table, pidx)
    return out.reshape(batch, maxlen, embed)


# speedup vs baseline: 1.5136x; 1.2628x over previous
import functools

import jax
import jax.numpy as jnp
from jax import lax
from jax.experimental import pallas as pl
from jax.experimental.pallas import tpu as pltpu
from jax.experimental.pallas import tpu_sc as plsc

_INFO = plsc.get_sparse_core_info()
_NC, _NS = _INFO.num_cores, _INFO.num_subcores
_NW = _NC * _NS

_NBUF = 8


def _make_kernel(batch, maxlen, embed):
    rows_w = batch // _NW

    mesh = plsc.VectorSubcoreMesh(core_axis_name="c", subcore_axis_name="s")

    @functools.partial(
        pl.kernel,
        out_type=jax.ShapeDtypeStruct((batch, maxlen, embed), jnp.float32),
        mesh=mesh,
        scratch_types=[
            pltpu.VMEM((rows_w, maxlen), jnp.int32),
            pltpu.VMEM((1, maxlen), jnp.int32),
            pltpu.VMEM((_NBUF, maxlen, embed), jnp.float32),
        ]
        + [pltpu.SemaphoreType.DMA] * _NBUF,
        compiler_params=pltpu.CompilerParams(use_tc_tiling_on_sc=False),
    )
    def k(x_hbm, tok_hbm, pos_hbm, pidx_hbm, out_hbm, idx_v, pidx_v, rows_v,
          *sems):
        wid = lax.axis_index("s") * _NC + lax.axis_index("c")
        base = wid * rows_w
        pltpu.sync_copy(x_hbm.at[pl.ds(base, rows_w)], idx_v)
        pltpu.sync_copy(pidx_hbm, pidx_v)

        @pl.loop(0, rows_w, step=_NBUF)
        def group(g):
            toks = []
            for b in range(_NBUF):
                toks.append(pltpu.async_copy(
                    tok_hbm.at[idx_v.at[g + b]], rows_v.at[b], sems[b]))
            poss = []
            for b in range(_NBUF):
                toks[b].wait()
                poss.append(pltpu.async_copy(
                    pos_hbm.at[pidx_v.at[0]], rows_v.at[b], sems[b],
                    add=True))
            outs = []
            for b in range(_NBUF):
                poss[b].wait()
                outs.append(pltpu.async_copy(
                    rows_v.at[b], out_hbm.at[base + g + b], sems[b]))
            for b in range(_NBUF):
                outs[b].wait()

    return k


def kernel(x, token_table, pos_table):
    batch, maxlen = x.shape
    embed = token_table.shape[-1]
    pidx = jnp.arange(maxlen, dtype=jnp.int32).reshape(1, maxlen)
    k = _make_kernel(batch, maxlen, embed)
    return k(x.astype(jnp.int32), token_table, pos_table, pidx)


# speedup vs baseline: 1.5173x; 1.0025x over previous
import functools

import jax
import jax.numpy as jnp
from jax import lax
from jax.experimental import pallas as pl
from jax.experimental.pallas import tpu as pltpu
from jax.experimental.pallas import tpu_sc as plsc

_INFO = plsc.get_sparse_core_info()
_NC, _NS = _INFO.num_cores, _INFO.num_subcores
_NW = _NC * _NS

_NBUF = 16


def _make_kernel(batch, maxlen, embed):
    rows_w = batch // _NW

    mesh = plsc.VectorSubcoreMesh(core_axis_name="c", subcore_axis_name="s")

    @functools.partial(
        pl.kernel,
        out_type=jax.ShapeDtypeStruct((batch, maxlen, embed), jnp.float32),
        mesh=mesh,
        scratch_types=[
            pltpu.VMEM((rows_w, maxlen), jnp.int32),
            pltpu.VMEM((1, maxlen), jnp.int32),
            pltpu.VMEM((_NBUF, maxlen, embed), jnp.float32),
        ]
        + [pltpu.SemaphoreType.DMA] * _NBUF,
        compiler_params=pltpu.CompilerParams(use_tc_tiling_on_sc=False),
    )
    def k(x_hbm, tok_hbm, pos_hbm, pidx_hbm, out_hbm, idx_v, pidx_v, rows_v,
          *sems):
        wid = lax.axis_index("s") * _NC + lax.axis_index("c")
        base = wid * rows_w
        pltpu.sync_copy(x_hbm.at[pl.ds(base, rows_w)], idx_v)
        pltpu.sync_copy(pidx_hbm, pidx_v)

        @pl.loop(0, rows_w, step=_NBUF)
        def group(g):
            toks = []
            for b in range(_NBUF):
                toks.append(pltpu.async_copy(
                    tok_hbm.at[idx_v.at[g + b]], rows_v.at[b], sems[b]))
            poss = []
            for b in range(_NBUF):
                toks[b].wait()
                poss.append(pltpu.async_copy(
                    pos_hbm.at[pidx_v.at[0]], rows_v.at[b], sems[b],
                    add=True))
            outs = []
            for b in range(_NBUF):
                poss[b].wait()
                outs.append(pltpu.async_copy(
                    rows_v.at[b], out_hbm.at[base + g + b], sems[b]))
            for b in range(_NBUF):
                outs[b].wait()

    return k


def kernel(x, token_table, pos_table):
    batch, maxlen = x.shape
    embed = token_table.shape[-1]
    pidx = jnp.arange(maxlen, dtype=jnp.int32).reshape(1, maxlen)
    k = _make_kernel(batch, maxlen, embed)
    return k(x.astype(jnp.int32), token_table, pos_table, pidx)
